# baseline (speedup 1.0000x reference)
import functools

import jax
import jax.numpy as jnp
from jax import lax
from jax.experimental import pallas as pl
from jax.experimental.pallas import tpu as pltpu
from jax.experimental.pallas import tpu_sc as plsc

BATCH = 16384
DIM = 32
NUM_CORES = 2
NUM_SUBCORES = 16
NUM_WORKERS = NUM_CORES * NUM_SUBCORES
BPW = BATCH // NUM_WORKERS
CHUNK = 128
NCHUNK = BPW // CHUNK
LANES = 16
LINE = 128
RPL = LINE // DIM


def _relmf_body(users_hbm, items_hbm, uemb_hbm, iemb_hbm,
                u_out, i_out, r_out,
                idx_v, lin_v, lines, u_pack, i_pack, r_v, sem):
    wid = lax.axis_index("s") * NUM_CORES + lax.axis_index("c")
    base = wid * BPW

    def stage_and_gather(src_idx_hbm, table_hbm):
        for j in range(NCHUNK):
            pltpu.sync_copy(src_idx_hbm.at[pl.ds(base + j * CHUNK, CHUNK)],
                            idx_v.at[j])

        def lin(j, carry):
            for k in range(CHUNK // LANES):
                s = pl.ds(k * LANES, LANES)
                lin_v[j, s] = lax.shift_right_logical(idx_v[j, s], 2)
            return carry

        lax.fori_loop(0, NCHUNK, lin, 0)

        copies = [
            pltpu.async_copy(table_hbm.at[lin_v.at[j]],
                             lines.at[pl.ds(j * CHUNK, CHUNK)], sem)
            for j in range(NCHUNK)
        ]
        for c in copies:
            c.wait()

    stage_and_gather(users_hbm, uemb_hbm)

    def u_group(g, carry):
        j, q0 = g // (CHUNK // LANES), (g % (CHUNK // LANES)) * LANES
        ofs = (idx_v[j, pl.ds(q0, LANES)] & (RPL - 1)) * DIM
        for k in range(LANES):
            r = g * LANES + k
            c = pl.multiple_of(ofs[k], DIM)
            p = g * (LANES // RPL) + k // RPL
            pc = (k % RPL) * DIM
            u_pack[p, pl.ds(pc, LANES)] = lines[r, pl.ds(c, LANES)]
            u_pack[p, pl.ds(pc + LANES, LANES)] = (
                lines[r, pl.ds(c + LANES, LANES)])
        return carry

    lax.fori_loop(0, BPW // LANES, u_group, 0)

    stage_and_gather(items_hbm, iemb_hbm)
    lane = lax.iota(jnp.int32, LANES)

    def i_group(g, carry):
        acc = jnp.zeros((LANES,), jnp.float32)
        j, q0 = g // (CHUNK // LANES), (g % (CHUNK // LANES)) * LANES
        ofs = (idx_v[j, pl.ds(q0, LANES)] & (RPL - 1)) * DIM
        for k in range(LANES):
            r = g * LANES + k
            c = pl.multiple_of(ofs[k], DIM)
            p = g * (LANES // RPL) + k // RPL
            pc = (k % RPL) * DIM
            ia = lines[r, pl.ds(c, LANES)]
            ib = lines[r, pl.ds(c + LANES, LANES)]
            i_pack[p, pl.ds(pc, LANES)] = ia
            i_pack[p, pl.ds(pc + LANES, LANES)] = ib
            ua = u_pack[p, pl.ds(pc, LANES)]
            ub = u_pack[p, pl.ds(pc + LANES, LANES)]
            s = jnp.sum(ua * ia + ub * ib)
            acc = jnp.where(lane == k, s, acc)
        r_v[pl.ds(pl.multiple_of(g * LANES, LANES), LANES)] = acc
        return carry

    lax.fori_loop(0, BPW // LANES, i_group, 0)

    pltpu.sync_copy(u_pack, u_out.at[pl.ds(wid * CHUNK, CHUNK)])
    pltpu.sync_copy(i_pack, i_out.at[pl.ds(wid * CHUNK, CHUNK)])
    pltpu.sync_copy(r_v, r_out.at[pl.ds(base, BPW)])


_relmf_sc = functools.partial(
    pl.kernel,
    out_type=(
        jax.ShapeDtypeStruct((BATCH // RPL, LINE), jnp.float32),
        jax.ShapeDtypeStruct((BATCH // RPL, LINE), jnp.float32),
        jax.ShapeDtypeStruct((BATCH,), jnp.float32),
    ),
    mesh=plsc.VectorSubcoreMesh(core_axis_name="c", subcore_axis_name="s"),
    compiler_params=pltpu.CompilerParams(needs_layout_passes=False),
    scratch_types=[
        pltpu.VMEM((NCHUNK, CHUNK), jnp.int32),
        pltpu.VMEM((NCHUNK, CHUNK), jnp.int32),
        pltpu.VMEM((BPW, LINE), jnp.float32),
        pltpu.VMEM((CHUNK, LINE), jnp.float32),
        pltpu.VMEM((CHUNK, LINE), jnp.float32),
        pltpu.VMEM((BPW,), jnp.float32),
        pltpu.SemaphoreType.DMA,
    ],
)(_relmf_body)


def kernel(users, items, user_embeddings, item_embeddings):
    u_tab = user_embeddings.reshape(-1, LINE)
    i_tab = item_embeddings.reshape(-1, LINE)
    u_pk, i_pk, r_hats = _relmf_sc(users, items, u_tab, i_tab)
    return (u_pk.reshape(BATCH, DIM), i_pk.reshape(BATCH, DIM), r_hats)
